# baseline (speedup 1.0000x reference)
import jax
import jax.numpy as jnp
from jax import lax
from jax.experimental import pallas as pl
from jax.experimental.pallas import tpu as pltpu

_E = 6400000
_C = 128
_R = _E // _C
_GRID = 25
_BR = _R // _GRID

_R0 = (13, 15, 26, 6)
_R1 = (17, 29, 16, 24)
_KS1 = 42
_KS2 = 0x1BD11BDA ^ 42


def _rotl(x, r):
    return (x << jnp.uint32(r)) | (x >> jnp.uint32(32 - r))


def _round(x0, x1, r):
    x0 = x0 + x1
    x1 = x0 ^ _rotl(x1, r)
    return x0, x1


def _threefry_bits(x):
    ks1 = jnp.uint32(_KS1)
    ks2 = jnp.uint32(_KS2)
    x1 = x + ks1
    x0 = x1
    x1 = x0 ^ _rotl(x1, _R0[0])
    for r in _R0[1:]:
        x0, x1 = _round(x0, x1, r)
    x0 = x0 + ks1
    x1 = x1 + jnp.uint32((_KS2 + 1) & 0xFFFFFFFF)
    for r in _R1:
        x0, x1 = _round(x0, x1, r)
    x0 = x0 + ks2
    x1 = x1 + jnp.uint32(2)
    for r in _R0:
        x0, x1 = _round(x0, x1, r)
    x1 = x1 + jnp.uint32(_KS1 + 3)
    for r in _R1:
        x0, x1 = _round(x0, x1, r)
    x0 = x0 + ks1
    x1 = x1 + jnp.uint32((_KS2 + 4) & 0xFFFFFFFF)
    for r in _R0:
        x0, x1 = _round(x0, x1, r)
    x0 = x0 + ks2
    x1 = x1 + jnp.uint32(5)
    return x0 ^ x1


def _body(kr_ref, vals_ref, idx_ref, ovals_ref, oidx_ref):
    oidx_ref[...] = idx_ref[...]

    kr = kr_ref[0]
    inv = 1.0 / kr
    g = pl.program_id(0)
    row = lax.broadcasted_iota(jnp.int32, (_BR, _C), 0)
    col = lax.broadcasted_iota(jnp.int32, (_BR, _C), 1)
    i = g * (_BR * _C) + row * _C + col
    bits = _threefry_bits(i.astype(jnp.uint32))
    u = lax.bitcast_convert_type(
        (bits >> jnp.uint32(9)) | jnp.uint32(0x3F800000), jnp.float32) - 1.0
    keep = (u + kr) >= 1.0
    ovals_ref[...] = jnp.where(keep, vals_ref[...] * inv, 0.0)


def kernel(adj_indices, adj_values, keepRate):
    assert adj_values.shape == (_E,) and adj_indices.shape == (2, _E)
    kr = jnp.asarray(keepRate, jnp.float32).reshape(1)
    vals2 = adj_values.reshape(_R, _C)
    idx3 = adj_indices.reshape(2, _R, _C)
    ovals, oidx = pl.pallas_call(
        _body,
        grid=(_GRID,),
        in_specs=[
            pl.BlockSpec(memory_space=pltpu.SMEM),
            pl.BlockSpec((_BR, _C), lambda g: (g, 0)),
            pl.BlockSpec((2, _BR, _C), lambda g: (0, g, 0)),
        ],
        out_specs=[
            pl.BlockSpec((_BR, _C), lambda g: (g, 0)),
            pl.BlockSpec((2, _BR, _C), lambda g: (0, g, 0)),
        ],
        out_shape=[
            jax.ShapeDtypeStruct((_R, _C), jnp.float32),
            jax.ShapeDtypeStruct((2, _R, _C), jnp.int32),
        ],
        compiler_params=pltpu.CompilerParams(
            dimension_semantics=("arbitrary",),
        ),
    )(kr, vals2, idx3)
    return oidx.reshape(2, _E), ovals.reshape(_E)
